# baseline (device time: 16231 ns/iter reference)
import jax
import jax.numpy as jnp
from jax import lax
from jax.experimental import pallas as pl
from jax.experimental.pallas import tpu as pltpu

N_DEV = 16


def kernel(x, w_mat):
    k_dim, m_blk = x.shape
    n_dim = w_mat.shape[1]
    blk = m_blk

    def body(x_ref, w_ref, out_ref, xblks_ref, send_sems, recv_sems):
        me = lax.axis_index("i")

        barrier_sem = pltpu.get_barrier_semaphore()
        for k in range(1, N_DEV):
            nbr = lax.rem(me + k, N_DEV)
            pl.semaphore_signal(
                barrier_sem, inc=1,
                device_id=(nbr,), device_id_type=pl.DeviceIdType.MESH,
            )
        pl.semaphore_wait(barrier_sem, N_DEV - 1)

        sends = []
        for k in range(1, N_DEV):
            dst = lax.rem(me + k, N_DEV)
            rdma = pltpu.make_async_remote_copy(
                src_ref=x_ref.at[pl.ds(dst * blk, blk), :],
                dst_ref=xblks_ref.at[me],
                send_sem=send_sems.at[dst],
                recv_sem=recv_sems.at[me],
                device_id=(dst,),
                device_id_type=pl.DeviceIdType.MESH,
            )
            rdma.start()
            sends.append(rdma)

        acc = jnp.dot(
            x_ref[pl.ds(me * blk, blk), :],
            w_ref[pl.ds(me * blk, blk), :],
            preferred_element_type=jnp.float32,
        )

        for k in range(1, N_DEV):
            j = lax.rem(me - k + N_DEV, N_DEV)
            recv = pltpu.make_async_remote_copy(
                src_ref=x_ref.at[pl.ds(0, blk), :],
                dst_ref=xblks_ref.at[j],
                send_sem=send_sems.at[j],
                recv_sem=recv_sems.at[j],
                device_id=(j,),
                device_id_type=pl.DeviceIdType.MESH,
            )
            recv.wait_recv()
            acc = acc + jnp.dot(
                xblks_ref[j],
                w_ref[pl.ds(j * blk, blk), :],
                preferred_element_type=jnp.float32,
            )

        for rdma in sends:
            rdma.wait_send()

        c = 0.7978845608028654
        out_ref[...] = 0.5 * acc * (1.0 + jnp.tanh(c * (acc + 0.044715 * acc * acc * acc)))

    return pl.pallas_call(
        body,
        out_shape=jax.ShapeDtypeStruct((blk, n_dim), jnp.float32),
        in_specs=[
            pl.BlockSpec(memory_space=pltpu.VMEM),
            pl.BlockSpec(memory_space=pltpu.VMEM),
        ],
        out_specs=pl.BlockSpec(memory_space=pltpu.VMEM),
        scratch_shapes=[
            pltpu.VMEM((N_DEV, blk, blk), jnp.float32),
            pltpu.SemaphoreType.DMA((N_DEV,)),
            pltpu.SemaphoreType.DMA((N_DEV,)),
        ],
        compiler_params=pltpu.CompilerParams(collective_id=0),
    )(x, w_mat)


# device time: 13697 ns/iter; 1.1850x vs baseline; 1.1850x over previous
import jax
import jax.numpy as jnp
from jax import lax
from jax.experimental import pallas as pl
from jax.experimental.pallas import tpu as pltpu

N_DEV = 16


def kernel(x, w_mat):
    k_dim, m_blk = x.shape
    n_dim = w_mat.shape[1]
    blk = m_blk

    def body(x_ref, w_ref, out_ref, xblks_ref, send_sems, recv_sems):
        me = lax.axis_index("i")

        barrier_sem = pltpu.get_barrier_semaphore()
        for k in range(1, N_DEV):
            nbr = lax.rem(me + k, N_DEV)
            pl.semaphore_signal(
                barrier_sem, inc=1,
                device_id=(nbr,), device_id_type=pl.DeviceIdType.MESH,
            )
        pl.semaphore_wait(barrier_sem, N_DEV - 1)

        sends = []
        for k in range(1, N_DEV):
            dst = lax.rem(me + k, N_DEV)
            rdma = pltpu.make_async_remote_copy(
                src_ref=x_ref.at[pl.ds(dst * blk, blk), :],
                dst_ref=xblks_ref.at[me],
                send_sem=send_sems.at[dst],
                recv_sem=recv_sems.at[me],
                device_id=(dst,),
                device_id_type=pl.DeviceIdType.MESH,
            )
            rdma.start()
            sends.append(rdma)

        xblks_ref[me] = x_ref[pl.ds(me * blk, blk), :]

        for k in range(1, N_DEV):
            j = lax.rem(me - k + N_DEV, N_DEV)
            recv = pltpu.make_async_remote_copy(
                src_ref=x_ref.at[pl.ds(0, blk), :],
                dst_ref=xblks_ref.at[j],
                send_sem=send_sems.at[j],
                recv_sem=recv_sems.at[j],
                device_id=(j,),
                device_id_type=pl.DeviceIdType.MESH,
            )
            recv.wait_recv()

        xrow = jnp.transpose(xblks_ref[...], (1, 0, 2)).reshape(blk, k_dim)
        acc = jnp.dot(xrow, w_ref[...], preferred_element_type=jnp.float32)

        for rdma in sends:
            rdma.wait_send()

        c = 0.7978845608028654
        out_ref[...] = 0.5 * acc * (1.0 + jnp.tanh(c * (acc + 0.044715 * acc * acc * acc)))

    return pl.pallas_call(
        body,
        out_shape=jax.ShapeDtypeStruct((blk, n_dim), jnp.float32),
        in_specs=[
            pl.BlockSpec(memory_space=pltpu.VMEM),
            pl.BlockSpec(memory_space=pltpu.VMEM),
        ],
        out_specs=pl.BlockSpec(memory_space=pltpu.VMEM),
        scratch_shapes=[
            pltpu.VMEM((N_DEV, blk, blk), jnp.float32),
            pltpu.SemaphoreType.DMA((N_DEV,)),
            pltpu.SemaphoreType.DMA((N_DEV,)),
        ],
        compiler_params=pltpu.CompilerParams(collective_id=0),
    )(x, w_mat)


# device time: 10705 ns/iter; 1.5162x vs baseline; 1.2795x over previous
import os

import jax
import jax.numpy as jnp
from jax import lax
from jax.experimental import pallas as pl
from jax.experimental.pallas import tpu as pltpu

N_DEV = 16
_VARIANT = os.environ.get("KVARIANT", "full")


def kernel(x, w_mat):
    k_dim, m_blk = x.shape
    n_dim = w_mat.shape[1]
    blk = m_blk

    def body(x_ref, w_ref, out_ref, xblks_ref, send_sems, recv_sems):
        me = lax.axis_index("i")

        barrier_sem = pltpu.get_barrier_semaphore()
        for k in range(1, N_DEV):
            nbr = lax.rem(me + k, N_DEV)
            pl.semaphore_signal(
                barrier_sem, inc=1,
                device_id=(nbr,), device_id_type=pl.DeviceIdType.MESH,
            )
        pl.semaphore_wait(barrier_sem, N_DEV - 1)

        do_comm = _VARIANT in ("full", "nocompute")
        do_compute = _VARIANT in ("full", "nocomm")

        sends = []
        if do_comm:
            for k in range(1, N_DEV):
                dst = lax.rem(me + k, N_DEV)
                rdma = pltpu.make_async_remote_copy(
                    src_ref=x_ref.at[pl.ds(dst * blk, blk), :],
                    dst_ref=xblks_ref.at[me],
                    send_sem=send_sems.at[dst],
                    recv_sem=recv_sems.at[me],
                    device_id=(dst,),
                    device_id_type=pl.DeviceIdType.MESH,
                )
                rdma.start()
                sends.append(rdma)

        xblks_ref[me] = x_ref[pl.ds(me * blk, blk), :]

        if do_comm:
            for k in range(1, N_DEV):
                j = lax.rem(me - k + N_DEV, N_DEV)
                recv = pltpu.make_async_remote_copy(
                    src_ref=x_ref.at[pl.ds(0, blk), :],
                    dst_ref=xblks_ref.at[j],
                    send_sem=send_sems.at[j],
                    recv_sem=recv_sems.at[j],
                    device_id=(j,),
                    device_id_type=pl.DeviceIdType.MESH,
                )
                recv.wait_recv()

        if do_compute:
            xrow = jnp.transpose(xblks_ref[...], (1, 0, 2)).reshape(blk, k_dim)
            acc = jnp.dot(xrow, w_ref[...], preferred_element_type=jnp.float32)

        for rdma in sends:
            rdma.wait_send()

        if do_compute:
            c = 0.7978845608028654
            out_ref[...] = 0.5 * acc * (1.0 + jnp.tanh(c * (acc + 0.044715 * acc * acc * acc)))
        else:
            out_ref[...] = jnp.zeros((blk, n_dim), jnp.float32)

    return pl.pallas_call(
        body,
        out_shape=jax.ShapeDtypeStruct((blk, n_dim), jnp.float32),
        in_specs=[
            pl.BlockSpec(memory_space=pltpu.VMEM),
            pl.BlockSpec(memory_space=pltpu.VMEM),
        ],
        out_specs=pl.BlockSpec(memory_space=pltpu.VMEM),
        scratch_shapes=[
            pltpu.VMEM((N_DEV, blk, blk), jnp.float32),
            pltpu.SemaphoreType.DMA((N_DEV,)),
            pltpu.SemaphoreType.DMA((N_DEV,)),
        ],
        compiler_params=pltpu.CompilerParams(collective_id=0),
    )(x, w_mat)


# device time: 4460 ns/iter; 3.6392x vs baseline; 2.4002x over previous
import os

import jax
import jax.numpy as jnp
from jax import lax
from jax.experimental import pallas as pl
from jax.experimental.pallas import tpu as pltpu

N_DEV = 16
_VARIANT = os.environ.get("KVARIANT", "full")


def kernel(x, w_mat):
    k_dim, m_blk = x.shape
    n_dim = w_mat.shape[1]
    blk = m_blk

    def body(x_ref, w_ref, out_ref, xblks_ref, send_sems, recv_sems):
        me = lax.axis_index("i")

        if _VARIANT != "empty":
            barrier_sem = pltpu.get_barrier_semaphore()
            for k in range(1, N_DEV):
                nbr = lax.rem(me + k, N_DEV)
                pl.semaphore_signal(
                    barrier_sem, inc=1,
                    device_id=(nbr,), device_id_type=pl.DeviceIdType.MESH,
                )
            pl.semaphore_wait(barrier_sem, N_DEV - 1)

        do_comm = _VARIANT in ("full", "nocompute")
        do_compute = _VARIANT in ("full", "nocomm")

        sends = []
        if do_comm:
            for k in range(1, N_DEV):
                dst = lax.rem(me + k, N_DEV)
                rdma = pltpu.make_async_remote_copy(
                    src_ref=x_ref.at[pl.ds(dst * blk, blk), :],
                    dst_ref=xblks_ref.at[me],
                    send_sem=send_sems.at[dst],
                    recv_sem=recv_sems.at[me],
                    device_id=(dst,),
                    device_id_type=pl.DeviceIdType.MESH,
                )
                rdma.start()
                sends.append(rdma)

        xblks_ref[me] = x_ref[pl.ds(me * blk, blk), :]

        if do_comm:
            for k in range(1, N_DEV):
                j = lax.rem(me - k + N_DEV, N_DEV)
                recv = pltpu.make_async_remote_copy(
                    src_ref=x_ref.at[pl.ds(0, blk), :],
                    dst_ref=xblks_ref.at[j],
                    send_sem=send_sems.at[j],
                    recv_sem=recv_sems.at[j],
                    device_id=(j,),
                    device_id_type=pl.DeviceIdType.MESH,
                )
                recv.wait_recv()

        if do_compute:
            xrow = jnp.transpose(xblks_ref[...], (1, 0, 2)).reshape(blk, k_dim)
            acc = jnp.dot(xrow, w_ref[...], preferred_element_type=jnp.float32)

        for rdma in sends:
            rdma.wait_send()

        if do_compute:
            c = 0.7978845608028654
            out_ref[...] = 0.5 * acc * (1.0 + jnp.tanh(c * (acc + 0.044715 * acc * acc * acc)))
        else:
            out_ref[...] = jnp.zeros((blk, n_dim), jnp.float32)

    return pl.pallas_call(
        body,
        out_shape=jax.ShapeDtypeStruct((blk, n_dim), jnp.float32),
        in_specs=[
            pl.BlockSpec(memory_space=pltpu.VMEM),
            pl.BlockSpec(memory_space=pltpu.VMEM),
        ],
        out_specs=pl.BlockSpec(memory_space=pltpu.VMEM),
        scratch_shapes=[
            pltpu.VMEM((N_DEV, blk, blk), jnp.float32),
            pltpu.SemaphoreType.DMA((N_DEV,)),
            pltpu.SemaphoreType.DMA((N_DEV,)),
        ],
        compiler_params=(
            pltpu.CompilerParams()
            if _VARIANT == "empty"
            else pltpu.CompilerParams(collective_id=0)
        ),
    )(x, w_mat)
